# baseline (speedup 1.0000x reference)
import functools

import jax
import jax.numpy as jnp
from jax import lax
from jax.experimental import pallas as pl
from jax.experimental.pallas import tpu as pltpu

E = 8
D = 2048
TOKENS = 8192

_BM = 512
_BK = 256
_NM = TOKENS // _BM
_NK = D // _BK


def _top2(sp):
    iota = lax.broadcasted_iota(jnp.int32, (1, E), 1)
    m1 = jnp.max(sp)
    i1 = jnp.min(jnp.where(sp == m1, iota, E))
    sp2 = jnp.where(iota == i1, -jnp.inf, sp)
    m2 = jnp.max(sp2)
    i2 = jnp.min(jnp.where(sp2 == m2, iota, E))
    e2 = jnp.exp(m2 - m1)
    denom = 1.0 + e2
    p0 = 1.0 / denom
    p1 = e2 / denom
    return i1, i2, p0, p1


def _gating_body(sp_ref, idx_ref, scores_ref):
    sp = sp_ref[...]
    i1, i2, p0, p1 = _top2(sp)
    iota = lax.broadcasted_iota(jnp.int32, (1, E), 1)
    idx_ref[...] = jnp.where(iota == 0, i1, jnp.where(iota == 1, i2, 0))
    scores_ref[...] = (
        jnp.where(iota == i1, p0, 0.0) + jnp.where(iota == i2, p1, 0.0)
    ).astype(jnp.float32)


def _fused_body(idx_ref, sp_ref, x_ref, w0_ref, w1_ref, b0_ref, b1_ref,
                out_ref, wc_scr):
    del idx_ref
    m = pl.program_id(0)
    k = pl.program_id(1)
    _, _, p0, p1 = _top2(sp_ref[...])

    @pl.when(m == 0)
    def _():
        wc_scr[pl.ds(k * _BK, _BK), :] = (
            p0 * w0_ref[0] + p1 * w1_ref[0]).astype(jnp.bfloat16)

    part = jnp.dot(x_ref[...].astype(jnp.bfloat16),
                   wc_scr[pl.ds(k * _BK, _BK), :],
                   preferred_element_type=jnp.float32)

    @pl.when(k == 0)
    def _():
        out_ref[...] = p0 * b0_ref[0] + p1 * b1_ref[0] + part

    @pl.when(k > 0)
    def _():
        out_ref[...] += part


@jax.jit
def kernel(x, W, b, scaling_params):
    sp = scaling_params.reshape(1, E)
    b3 = b.reshape(E, 1, D)

    idx_pad, scores = pl.pallas_call(
        _gating_body,
        out_shape=[
            jax.ShapeDtypeStruct((1, E), jnp.int32),
            jax.ShapeDtypeStruct((1, E), jnp.float32),
        ],
    )(sp)
    idx = idx_pad[0, :2]

    out = pl.pallas_call(
        _fused_body,
        grid_spec=pltpu.PrefetchScalarGridSpec(
            num_scalar_prefetch=1,
            grid=(_NM, _NK),
            in_specs=[
                pl.BlockSpec((1, E), lambda m, k, idx: (0, 0)),
                pl.BlockSpec((_BM, _BK), lambda m, k, idx: (m, k)),
                pl.BlockSpec(
                    (1, _BK, D),
                    lambda m, k, idx: (idx[0], jnp.where(m == 0, k, _NK - 1), 0)),
                pl.BlockSpec(
                    (1, _BK, D),
                    lambda m, k, idx: (idx[1], jnp.where(m == 0, k, _NK - 1), 0)),
                pl.BlockSpec((1, 1, D), lambda m, k, idx: (idx[0], 0, 0)),
                pl.BlockSpec((1, 1, D), lambda m, k, idx: (idx[1], 0, 0)),
            ],
            out_specs=pl.BlockSpec((_BM, D), lambda m, k, idx: (m, 0)),
            scratch_shapes=[pltpu.VMEM((D, D), jnp.bfloat16)],
        ),
        out_shape=jax.ShapeDtypeStruct((TOKENS, D), jnp.float32),
        compiler_params=pltpu.CompilerParams(
            dimension_semantics=("arbitrary", "arbitrary")),
    )(idx, sp, x, W, W, b3, b3)

    return out, scores.reshape(E)


# speedup vs baseline: 2.3247x; 2.3247x over previous
import functools

import jax
import jax.numpy as jnp
from jax import lax
from jax.experimental import pallas as pl
from jax.experimental.pallas import tpu as pltpu

E = 8
D = 2048
TOKENS = 8192

_BM = 512
_NM = TOKENS // _BM


def _top2(sp):
    iota = lax.broadcasted_iota(jnp.int32, (1, E), 1)
    m1 = jnp.max(sp)
    i1 = jnp.min(jnp.where(sp == m1, iota, E))
    sp2 = jnp.where(iota == i1, -jnp.inf, sp)
    m2 = jnp.max(sp2)
    i2 = jnp.min(jnp.where(sp2 == m2, iota, E))
    e2 = jnp.exp(m2 - m1)
    denom = 1.0 + e2
    p0 = 1.0 / denom
    p1 = e2 / denom
    return i1, i2, p0, p1


def _gating_body(sp_ref, idx_ref, scores_ref):
    sp = sp_ref[...]
    i1, i2, p0, p1 = _top2(sp)
    iota = lax.broadcasted_iota(jnp.int32, (1, E), 1)
    idx_ref[...] = jnp.where(iota == 0, i1, jnp.where(iota == 1, i2, 0))
    scores_ref[...] = (
        jnp.where(iota == i1, p0, 0.0) + jnp.where(iota == i2, p1, 0.0)
    ).astype(jnp.float32)


def _fused_body(idx_ref, sp_ref, x_ref, w0_ref, w1_ref, b0_ref, b1_ref,
                out_ref, wc_scr):
    del idx_ref
    m = pl.program_id(0)
    _, _, p0, p1 = _top2(sp_ref[...])

    @pl.when(m == 0)
    def _():
        wc_scr[...] = (p0 * w0_ref[0] + p1 * w1_ref[0]).astype(jnp.bfloat16)

    acc = jnp.dot(x_ref[...].astype(jnp.bfloat16), wc_scr[...],
                  preferred_element_type=jnp.float32)
    out_ref[...] = acc + (p0 * b0_ref[0] + p1 * b1_ref[0])


@jax.jit
def kernel(x, W, b, scaling_params):
    sp = scaling_params.reshape(1, E)
    b3 = b.reshape(E, 1, D)

    idx_pad, scores = pl.pallas_call(
        _gating_body,
        out_shape=[
            jax.ShapeDtypeStruct((1, E), jnp.int32),
            jax.ShapeDtypeStruct((1, E), jnp.float32),
        ],
    )(sp)
    idx = idx_pad[0, :2]

    out = pl.pallas_call(
        _fused_body,
        grid_spec=pltpu.PrefetchScalarGridSpec(
            num_scalar_prefetch=1,
            grid=(_NM,),
            in_specs=[
                pl.BlockSpec((1, E), lambda m, idx: (0, 0)),
                pl.BlockSpec((_BM, D), lambda m, idx: (m, 0)),
                pl.BlockSpec((1, D, D), lambda m, idx: (idx[0], 0, 0)),
                pl.BlockSpec((1, D, D), lambda m, idx: (idx[1], 0, 0)),
                pl.BlockSpec((1, 1, D), lambda m, idx: (idx[0], 0, 0)),
                pl.BlockSpec((1, 1, D), lambda m, idx: (idx[1], 0, 0)),
            ],
            out_specs=pl.BlockSpec((_BM, D), lambda m, idx: (m, 0)),
            scratch_shapes=[pltpu.VMEM((D, D), jnp.bfloat16)],
        ),
        out_shape=jax.ShapeDtypeStruct((TOKENS, D), jnp.float32),
        compiler_params=pltpu.CompilerParams(
            dimension_semantics=("arbitrary",)),
    )(idx, sp, x, W, W, b3, b3)

    return out, scores.reshape(E)
